# Initial kernel scaffold; baseline (speedup 1.0000x reference)
import jax
import jax.numpy as jnp
from jax.experimental import pallas as pl


def kernel(x, edge_index, edge_attr, batch, We, be, W1, b1, Wc, bc):
    raise NotImplementedError("write your pallas kernel here")


# speedup vs baseline: 2.5274x; 2.5274x over previous
import functools

import jax
import jax.numpy as jnp
from jax import lax
from jax.experimental import pallas as pl
from jax.experimental.pallas import tpu as pltpu
from jax.experimental.pallas import tpu_sc as plsc

N = 10000
E = 320000
D = 128
DE = 16
H = 128
G = 128

NC = 2
NS = 16
NW = NC * NS
EP = E // NW
CH = 80
NCHUNK = EP // CH
N_PAD = 10240
ROWS_PER_TILE = N_PAD // NS
ZR = 128


_BE = 4000


def _emb_body(ea_ref, we_ref, be_ref, out_ref):
    out_ref[...] = (
        jnp.dot(ea_ref[...], we_ref[...], preferred_element_type=jnp.float32)
        + be_ref[...]
    )


def _edge_emb(edge_attr, We, be2d):
    return pl.pallas_call(
        _emb_body,
        grid=(E // _BE,),
        in_specs=[
            pl.BlockSpec((_BE, DE), lambda i: (i, 0)),
            pl.BlockSpec((DE, D), lambda i: (0, 0)),
            pl.BlockSpec((1, D), lambda i: (0, 0)),
        ],
        out_specs=pl.BlockSpec((_BE, D), lambda i: (i, 0)),
        out_shape=jax.ShapeDtypeStruct((E, D), jnp.float32),
    )(edge_attr, We, be2d)


def _sc_edge_body(x_hbm, emb_hbm, src_hbm, dst_hbm, out_hbm,
                  agg_sh, sidx, didx, embv, xv, zbuf, sem):
    cid = lax.axis_index("c")
    sid = lax.axis_index("s")
    wid = cid * NS + sid
    base = wid * EP

    zeros16 = jnp.zeros((16,), jnp.float32)

    def _zrow(r, _):
        for c in range(D // 16):
            zbuf[r, pl.ds(c * 16, 16)] = zeros16
        return 0

    lax.fori_loop(0, ZR, _zrow, 0)
    for k in range(ROWS_PER_TILE // ZR):
        pltpu.sync_copy(zbuf, agg_sh.at[pl.ds(sid * ROWS_PER_TILE + k * ZR, ZR)])
    plsc.subcore_barrier()

    def _chunk(j, _):
        off = base + j * CH
        pltpu.sync_copy(src_hbm.at[pl.ds(off, CH)], sidx)
        pltpu.sync_copy(dst_hbm.at[pl.ds(off, CH)], didx)
        pltpu.async_copy(x_hbm.at[sidx], xv, sem).wait()
        pltpu.sync_copy(emb_hbm.at[pl.ds(off, CH)], embv)

        def _row(r, _):
            for c in range(D // 16):
                sl = pl.ds(c * 16, 16)
                embv[r, sl] = jnp.maximum(xv[r, sl] + embv[r, sl], 0.0)
            return 0

        lax.fori_loop(0, CH, _row, 0)
        pltpu.sync_copy(embv, agg_sh.at[didx], add=True)
        return 0

    lax.fori_loop(0, NCHUNK, _chunk, 0)
    plsc.subcore_barrier()

    row0 = sid * ROWS_PER_TILE
    pltpu.sync_copy(agg_sh.at[pl.ds(row0, ROWS_PER_TILE)],
                    out_hbm.at[cid, pl.ds(row0, ROWS_PER_TILE)])


@functools.cache
def _make_sc_edge_kernel():
    return functools.partial(
        pl.kernel,
        out_type=jax.ShapeDtypeStruct((NC, N_PAD, D), jnp.float32),
        mesh=plsc.VectorSubcoreMesh(core_axis_name="c", subcore_axis_name="s",
                                    num_cores=NC, num_subcores=NS),
        scratch_types=[
            pltpu.VMEM_SHARED((N_PAD, D), jnp.float32),
            pltpu.VMEM((CH,), jnp.int32),
            pltpu.VMEM((CH,), jnp.int32),
            pltpu.VMEM((CH, D), jnp.float32),
            pltpu.VMEM((CH, D), jnp.float32),
            pltpu.VMEM((ZR, D), jnp.float32),
            pltpu.SemaphoreType.DMA,
        ],
    )(_sc_edge_body)


_BN = 2000


def _node_body(x_ref, p0_ref, p1_ref, b_ref, w1_ref, b1_ref, wc_ref, bc_ref,
               out_ref, pooled_acc, cnt_acc):
    i = pl.program_id(0)

    @pl.when(i == 0)
    def _():
        pooled_acc[...] = jnp.zeros_like(pooled_acc)
        cnt_acc[...] = jnp.zeros_like(cnt_acc)

    t = x_ref[...] + p0_ref[...] + p1_ref[...]
    h = jnp.maximum(
        jnp.dot(t, w1_ref[...], preferred_element_type=jnp.float32) + b1_ref[...],
        0.0,
    )
    bvec = b_ref[0, 0, :]
    gids = lax.broadcasted_iota(jnp.int32, (_BN, G), 1)
    onehot = jnp.where(bvec[:, None] == gids, 1.0, 0.0)
    pooled_acc[...] += lax.dot_general(
        onehot, h, (((0,), (0,)), ((), ())), preferred_element_type=jnp.float32)
    cnt_acc[...] += lax.dot_general(
        onehot, jnp.ones((_BN, G), jnp.float32), (((0,), (0,)), ((), ())),
        preferred_element_type=jnp.float32)

    @pl.when(i == pl.num_programs(0) - 1)
    def _():
        cnt = jnp.maximum(cnt_acc[...], 1.0)
        logits = jnp.dot(pooled_acc[...], wc_ref[...],
                         preferred_element_type=jnp.float32)
        out_ref[...] = logits / cnt[:, :2] + bc_ref[...]


def _node_pool(x, p0, p1, batch3d, W1, b12d, Wc, bc2d):
    return pl.pallas_call(
        _node_body,
        grid=(N // _BN,),
        in_specs=[
            pl.BlockSpec((_BN, D), lambda i: (i, 0)),
            pl.BlockSpec((_BN, D), lambda i: (i, 0)),
            pl.BlockSpec((_BN, D), lambda i: (i, 0)),
            pl.BlockSpec((1, 1, _BN), lambda i: (i, 0, 0)),
            pl.BlockSpec((D, H), lambda i: (0, 0)),
            pl.BlockSpec((1, H), lambda i: (0, 0)),
            pl.BlockSpec((H, 2), lambda i: (0, 0)),
            pl.BlockSpec((1, 2), lambda i: (0, 0)),
        ],
        out_specs=pl.BlockSpec((G, 2), lambda i: (0, 0)),
        out_shape=jax.ShapeDtypeStruct((G, 2), jnp.float32),
        scratch_shapes=[
            pltpu.VMEM((G, H), jnp.float32),
            pltpu.VMEM((G, G), jnp.float32),
        ],
    )(x, p0, p1, batch3d, W1, b12d, Wc, bc2d)


def kernel(x, edge_index, edge_attr, batch, We, be, W1, b1, Wc, bc):
    emb = _edge_emb(edge_attr, We, be.reshape(1, D))
    src = edge_index[0]
    dst = edge_index[1]
    partials = _make_sc_edge_kernel()(x, emb, src, dst)[:, :N]
    batch3d = batch.reshape(N // _BN, 1, _BN)
    return _node_pool(x, partials[0], partials[1], batch3d,
                      W1, b1.reshape(1, H), Wc, bc.reshape(1, 2))


# speedup vs baseline: 3.7135x; 1.4693x over previous
import functools

import jax
import jax.numpy as jnp
from jax import lax
from jax.experimental import pallas as pl
from jax.experimental.pallas import tpu as pltpu
from jax.experimental.pallas import tpu_sc as plsc

N = 10000
E = 320000
D = 128
DE = 16
H = 128
G = 128

NC = 2
NS = 16
NW = NC * NS
EP = E // NW
PASSES = 5
EPP = EP // PASSES
CH = 40
NCHUNK = EPP // CH
N_PAD = 10240
ROWS_PER_TILE = N_PAD // NS
ZR = CH


_BE = 4000


def _emb_body(ea_ref, we_ref, be_ref, out_ref):
    out_ref[...] = (
        jnp.dot(ea_ref[...], we_ref[...], preferred_element_type=jnp.float32,
                precision=lax.Precision.HIGHEST)
        + be_ref[...]
    )


def _edge_emb(edge_attr, We, be2d):
    return pl.pallas_call(
        _emb_body,
        grid=(E // _BE,),
        in_specs=[
            pl.BlockSpec((_BE, DE), lambda i: (i, 0)),
            pl.BlockSpec((DE, D), lambda i: (0, 0)),
            pl.BlockSpec((1, D), lambda i: (0, 0)),
        ],
        out_specs=pl.BlockSpec((_BE, D), lambda i: (i, 0)),
        out_shape=jax.ShapeDtypeStruct((E, D), jnp.float32),
    )(edge_attr, We, be2d)


def _sc_edge_body(x_hbm, emb_hbm, src_hbm, dst_hbm, out_hbm,
                  agg_sh, sidx_all, didx_all,
                  embv0, embv1, embv2, xv0, xv1, xv2,
                  sem_e0, sem_e1, sem_e2, sem_g0, sem_g1, sem_g2,
                  sem_c0, sem_c1, sem_c2):
    cid = lax.axis_index("c")
    sid = lax.axis_index("s")
    wid = cid * NS + sid
    base = wid * EP

    embv = (embv0, embv1, embv2)
    xv = (xv0, xv1, xv2)
    sem_e = (sem_e0, sem_e1, sem_e2)
    sem_g = (sem_g0, sem_g1, sem_g2)
    sem_c = (sem_c0, sem_c1, sem_c2)

    def _issue_inputs(pbase, j, p):
        pltpu.async_copy(emb_hbm.at[pl.ds(pbase + j * CH, CH)], embv[p], sem_e[p])
        pltpu.async_copy(x_hbm.at[sidx_all.at[j]], xv[p], sem_g[p])

    def _wait_inputs(p):
        pltpu.make_async_copy(emb_hbm.at[pl.ds(base, CH)], embv[p], sem_e[p]).wait()
        pltpu.make_async_copy(x_hbm.at[sidx_all.at[0]], xv[p], sem_g[p]).wait()

    def _issue_scatter(j, p):
        pltpu.async_copy(embv[p], agg_sh.at[didx_all.at[j]], sem_c[p], add=True)

    def _wait_scatter(p):
        pltpu.make_async_copy(embv[p], agg_sh.at[didx_all.at[0]], sem_c[p]).wait()

    def _compute(p):
        ev = embv[p]
        gv = xv[p]

        @plsc.parallel_loop(0, CH, step=1, unroll=2)
        def _row(r):
            for c in range(D // 16):
                sl = pl.ds(c * 16, 16)
                ev[r, sl] = jnp.maximum(gv[r, sl] + ev[r, sl], 0.0)

    NT = (NCHUNK - 2) // 3

    for q in range(PASSES):
        pbase = base + q * EPP
        pltpu.sync_copy(src_hbm.at[wid, q], sidx_all)
        pltpu.sync_copy(dst_hbm.at[wid, q], didx_all)
        _issue_inputs(pbase, 0, 0)
        _issue_inputs(pbase, 1, 1)

        if q == 0:
            zeros16 = jnp.zeros((16,), jnp.float32)

            def _zrow(r, _):
                for c in range(D // 16):
                    embv2[r, pl.ds(c * 16, 16)] = zeros16
                return 0

            lax.fori_loop(0, ZR, _zrow, 0)
            for k in range(ROWS_PER_TILE // ZR):
                pltpu.sync_copy(
                    embv2, agg_sh.at[pl.ds(sid * ROWS_PER_TILE + k * ZR, ZR)])
            plsc.subcore_barrier()

        def _step(j, p, guard_first):
            _wait_inputs(p)
            _compute(p)
            if guard_first:
                pl.when(j >= 1)(lambda: _wait_scatter((p + 2) % 3))
            else:
                _wait_scatter((p + 2) % 3)
            _issue_inputs(pbase, j + 2, (p + 2) % 3)
            _issue_scatter(j, p)

        def _triple(t, _):
            j0 = 3 * t
            _step(j0, 0, True)
            _step(j0 + 1, 1, False)
            _step(j0 + 2, 2, False)
            return 0

        lax.fori_loop(0, NT, _triple, 0)

        for j in range(3 * NT, NCHUNK):
            p = j % 3
            _wait_inputs(p)
            _compute(p)
            _wait_scatter((p + 2) % 3)
            _issue_scatter(j, p)
        _wait_scatter((NCHUNK - 1) % 3)
    plsc.subcore_barrier()

    row0 = sid * ROWS_PER_TILE
    pltpu.sync_copy(agg_sh.at[pl.ds(row0, ROWS_PER_TILE)],
                    out_hbm.at[cid, pl.ds(row0, ROWS_PER_TILE)])


@functools.cache
def _make_sc_edge_kernel():
    return functools.partial(
        pl.kernel,
        out_type=jax.ShapeDtypeStruct((NC, N_PAD, D), jnp.float32),
        mesh=plsc.VectorSubcoreMesh(core_axis_name="c", subcore_axis_name="s",
                                    num_cores=NC, num_subcores=NS),
        scratch_types=[
            pltpu.VMEM_SHARED((N_PAD, D), jnp.float32),
            pltpu.VMEM((NCHUNK, CH), jnp.int32),
            pltpu.VMEM((NCHUNK, CH), jnp.int32),
            pltpu.VMEM((CH, D), jnp.float32),
            pltpu.VMEM((CH, D), jnp.float32),
            pltpu.VMEM((CH, D), jnp.float32),
            pltpu.VMEM((CH, D), jnp.float32),
            pltpu.VMEM((CH, D), jnp.float32),
            pltpu.VMEM((CH, D), jnp.float32),
            pltpu.SemaphoreType.DMA,
            pltpu.SemaphoreType.DMA,
            pltpu.SemaphoreType.DMA,
            pltpu.SemaphoreType.DMA,
            pltpu.SemaphoreType.DMA,
            pltpu.SemaphoreType.DMA,
            pltpu.SemaphoreType.DMA,
            pltpu.SemaphoreType.DMA,
            pltpu.SemaphoreType.DMA,
        ],
    )(_sc_edge_body)


_BN = 2000


def _node_body(x_ref, p0_ref, p1_ref, b_ref, w1_ref, b1_ref, wc_ref, bc_ref,
               out_ref, pooled_acc, cnt_acc):
    i = pl.program_id(0)

    @pl.when(i == 0)
    def _():
        pooled_acc[...] = jnp.zeros_like(pooled_acc)
        cnt_acc[...] = jnp.zeros_like(cnt_acc)

    t = x_ref[...] + p0_ref[...] + p1_ref[...]
    h = jnp.maximum(
        jnp.dot(t, w1_ref[...], preferred_element_type=jnp.float32,
                precision=lax.Precision.HIGHEST) + b1_ref[...],
        0.0,
    )
    bvec = b_ref[0, 0, :]
    gids = lax.broadcasted_iota(jnp.int32, (_BN, G), 1)
    onehot = jnp.where(bvec[:, None] == gids, 1.0, 0.0)
    pooled_acc[...] += lax.dot_general(
        onehot, h, (((0,), (0,)), ((), ())), preferred_element_type=jnp.float32,
        precision=lax.Precision.HIGHEST)
    cnt_acc[...] += lax.dot_general(
        onehot, jnp.ones((_BN, G), jnp.float32), (((0,), (0,)), ((), ())),
        preferred_element_type=jnp.float32)

    @pl.when(i == pl.num_programs(0) - 1)
    def _():
        cnt = jnp.maximum(cnt_acc[...], 1.0)
        logits = jnp.dot(pooled_acc[...], wc_ref[...],
                         preferred_element_type=jnp.float32,
                         precision=lax.Precision.HIGHEST)
        out_ref[...] = logits / cnt[:, :2] + bc_ref[...]


def _node_pool(x, p0, p1, batch3d, W1, b12d, Wc, bc2d):
    return pl.pallas_call(
        _node_body,
        grid=(N // _BN,),
        in_specs=[
            pl.BlockSpec((_BN, D), lambda i: (i, 0)),
            pl.BlockSpec((_BN, D), lambda i: (i, 0)),
            pl.BlockSpec((_BN, D), lambda i: (i, 0)),
            pl.BlockSpec((1, 1, _BN), lambda i: (i, 0, 0)),
            pl.BlockSpec((D, H), lambda i: (0, 0)),
            pl.BlockSpec((1, H), lambda i: (0, 0)),
            pl.BlockSpec((H, 2), lambda i: (0, 0)),
            pl.BlockSpec((1, 2), lambda i: (0, 0)),
        ],
        out_specs=pl.BlockSpec((G, 2), lambda i: (0, 0)),
        out_shape=jax.ShapeDtypeStruct((G, 2), jnp.float32),
        scratch_shapes=[
            pltpu.VMEM((G, H), jnp.float32),
            pltpu.VMEM((G, G), jnp.float32),
        ],
    )(x, p0, p1, batch3d, W1, b12d, Wc, bc2d)


def kernel(x, edge_index, edge_attr, batch, We, be, W1, b1, Wc, bc):
    emb = _edge_emb(edge_attr, We, be.reshape(1, D))
    src = edge_index[0].reshape(NW, PASSES, NCHUNK, CH)
    dst = edge_index[1].reshape(NW, PASSES, NCHUNK, CH)
    partials = _make_sc_edge_kernel()(x, emb, src, dst)[:, :N]
    batch3d = batch.reshape(N // _BN, 1, _BN)
    return _node_pool(x, partials[0], partials[1], batch3d,
                      W1, b1.reshape(1, H), Wc, bc.reshape(1, 2))


# speedup vs baseline: 3.8654x; 1.0409x over previous
import functools

import jax
import jax.numpy as jnp
from jax import lax
from jax.experimental import pallas as pl
from jax.experimental.pallas import tpu as pltpu
from jax.experimental.pallas import tpu_sc as plsc

N = 10000
E = 320000
D = 128
DE = 16
H = 128
G = 128

NC = 2
NS = 16
NW = NC * NS
EP = E // NW
PASSES = 2
EPP = EP // PASSES
CH = 40
NCHUNK = EPP // CH
N_PAD = 10240
ROWS_PER_TILE = N_PAD // NS
ZR = CH


_BE = 4000


def _emb_body(ea_ref, we_ref, be_ref, out_ref):
    out_ref[...] = (
        jnp.dot(ea_ref[...], we_ref[...], preferred_element_type=jnp.float32,
                precision=lax.Precision.HIGHEST)
        + be_ref[...]
    )


def _edge_emb(edge_attr, We, be2d):
    return pl.pallas_call(
        _emb_body,
        grid=(E // _BE,),
        in_specs=[
            pl.BlockSpec((_BE, DE), lambda i: (i, 0)),
            pl.BlockSpec((DE, D), lambda i: (0, 0)),
            pl.BlockSpec((1, D), lambda i: (0, 0)),
        ],
        out_specs=pl.BlockSpec((_BE, D), lambda i: (i, 0)),
        out_shape=jax.ShapeDtypeStruct((E, D), jnp.float32),
    )(edge_attr, We, be2d)


def _sc_edge_body(x_hbm, emb_hbm, src_hbm, dst_hbm, out_hbm,
                  agg_sh, sidx_all,
                  embv0, embv1, embv2, xv0, xv1, xv2, didx0, didx1, didx2,
                  sem_e0, sem_e1, sem_e2, sem_g0, sem_g1, sem_g2,
                  sem_c0, sem_c1, sem_c2, sem_d0, sem_d1, sem_d2):
    cid = lax.axis_index("c")
    sid = lax.axis_index("s")
    wid = cid * NS + sid
    base = wid * EP

    embv = (embv0, embv1, embv2)
    xv = (xv0, xv1, xv2)
    didx = (didx0, didx1, didx2)
    sem_e = (sem_e0, sem_e1, sem_e2)
    sem_g = (sem_g0, sem_g1, sem_g2)
    sem_c = (sem_c0, sem_c1, sem_c2)
    sem_d = (sem_d0, sem_d1, sem_d2)

    def _issue_inputs(pbase, j, p):
        pltpu.async_copy(emb_hbm.at[pl.ds(pbase + j * CH, CH)], embv[p], sem_e[p])
        pltpu.async_copy(x_hbm.at[sidx_all.at[pl.ds(j * CH, CH)]], xv[p], sem_g[p])
        pltpu.async_copy(dst_hbm.at[pl.ds(pbase + j * CH, CH)], didx[p], sem_d[p])

    def _wait_inputs(p):
        pltpu.make_async_copy(emb_hbm.at[pl.ds(base, CH)], embv[p], sem_e[p]).wait()
        pltpu.make_async_copy(x_hbm.at[sidx_all.at[pl.ds(0, CH)]], xv[p], sem_g[p]).wait()
        pltpu.make_async_copy(dst_hbm.at[pl.ds(base, CH)], didx[p], sem_d[p]).wait()

    def _issue_scatter(j, p):
        pltpu.async_copy(embv[p], agg_sh.at[didx[p]], sem_c[p], add=True)

    def _wait_scatter(p):
        pltpu.make_async_copy(embv[p], agg_sh.at[didx[p]], sem_c[p]).wait()

    def _compute(p):
        ev = embv[p]
        gv = xv[p]

        @plsc.parallel_loop(0, CH, step=1, unroll=2)
        def _row(r):
            for c in range(D // 16):
                sl = pl.ds(c * 16, 16)
                ev[r, sl] = jnp.maximum(gv[r, sl] + ev[r, sl], 0.0)

    NT = (NCHUNK - 2) // 3

    for q in range(PASSES):
        pbase = base + q * EPP
        pltpu.sync_copy(src_hbm.at[pl.ds(pbase, EPP)], sidx_all)
        _issue_inputs(pbase, 0, 0)
        _issue_inputs(pbase, 1, 1)

        if q == 0:
            zeros16 = jnp.zeros((16,), jnp.float32)

            def _zrow(r, _):
                for c in range(D // 16):
                    embv2[r, pl.ds(c * 16, 16)] = zeros16
                return 0

            lax.fori_loop(0, ZR, _zrow, 0)
            for k in range(ROWS_PER_TILE // ZR):
                pltpu.sync_copy(
                    embv2, agg_sh.at[pl.ds(sid * ROWS_PER_TILE + k * ZR, ZR)])
            plsc.subcore_barrier()

        def _step(j, p, guard_first):
            _wait_inputs(p)
            _compute(p)
            if guard_first:
                pl.when(j >= 1)(lambda: _wait_scatter((p + 2) % 3))
            else:
                _wait_scatter((p + 2) % 3)
            _issue_inputs(pbase, j + 2, (p + 2) % 3)
            _issue_scatter(j, p)

        def _triple(t, _):
            j0 = 3 * t
            _step(j0, 0, True)
            _step(j0 + 1, 1, False)
            _step(j0 + 2, 2, False)
            return 0

        lax.fori_loop(0, NT, _triple, 0)

        for j in range(3 * NT, NCHUNK):
            p = j % 3
            _wait_inputs(p)
            _compute(p)
            _wait_scatter((p + 2) % 3)
            _issue_scatter(j, p)
        _wait_scatter((NCHUNK - 1) % 3)
    plsc.subcore_barrier()

    row0 = sid * ROWS_PER_TILE
    pltpu.sync_copy(agg_sh.at[pl.ds(row0, ROWS_PER_TILE)],
                    out_hbm.at[cid, pl.ds(row0, ROWS_PER_TILE)])


@functools.cache
def _make_sc_edge_kernel():
    return functools.partial(
        pl.kernel,
        out_type=jax.ShapeDtypeStruct((NC, N_PAD, D), jnp.float32),
        mesh=plsc.VectorSubcoreMesh(core_axis_name="c", subcore_axis_name="s",
                                    num_cores=NC, num_subcores=NS),
        scratch_types=[
            pltpu.VMEM_SHARED((N_PAD, D), jnp.float32),
            pltpu.VMEM((EPP,), jnp.int32),
            pltpu.VMEM((CH, D), jnp.float32),
            pltpu.VMEM((CH, D), jnp.float32),
            pltpu.VMEM((CH, D), jnp.float32),
            pltpu.VMEM((CH, D), jnp.float32),
            pltpu.VMEM((CH, D), jnp.float32),
            pltpu.VMEM((CH, D), jnp.float32),
            pltpu.VMEM((CH,), jnp.int32),
            pltpu.VMEM((CH,), jnp.int32),
            pltpu.VMEM((CH,), jnp.int32),
            pltpu.SemaphoreType.DMA,
            pltpu.SemaphoreType.DMA,
            pltpu.SemaphoreType.DMA,
            pltpu.SemaphoreType.DMA,
            pltpu.SemaphoreType.DMA,
            pltpu.SemaphoreType.DMA,
            pltpu.SemaphoreType.DMA,
            pltpu.SemaphoreType.DMA,
            pltpu.SemaphoreType.DMA,
            pltpu.SemaphoreType.DMA,
            pltpu.SemaphoreType.DMA,
            pltpu.SemaphoreType.DMA,
        ],
    )(_sc_edge_body)


_BN = 2000


def _node_body(x_ref, p0_ref, p1_ref, b_ref, w1_ref, b1_ref, wc_ref, bc_ref,
               out_ref, pooled_acc, cnt_acc):
    i = pl.program_id(0)

    @pl.when(i == 0)
    def _():
        pooled_acc[...] = jnp.zeros_like(pooled_acc)
        cnt_acc[...] = jnp.zeros_like(cnt_acc)

    t = x_ref[...] + p0_ref[...] + p1_ref[...]
    h = jnp.maximum(
        jnp.dot(t, w1_ref[...], preferred_element_type=jnp.float32,
                precision=lax.Precision.HIGHEST) + b1_ref[...],
        0.0,
    )
    bvec = b_ref[0, 0, :]
    gids = lax.broadcasted_iota(jnp.int32, (_BN, G), 1)
    onehot = jnp.where(bvec[:, None] == gids, 1.0, 0.0)
    pooled_acc[...] += lax.dot_general(
        onehot, h, (((0,), (0,)), ((), ())), preferred_element_type=jnp.float32,
        precision=lax.Precision.HIGHEST)
    cnt_acc[...] += lax.dot_general(
        onehot, jnp.ones((_BN, G), jnp.float32), (((0,), (0,)), ((), ())),
        preferred_element_type=jnp.float32)

    @pl.when(i == pl.num_programs(0) - 1)
    def _():
        cnt = jnp.maximum(cnt_acc[...], 1.0)
        logits = jnp.dot(pooled_acc[...], wc_ref[...],
                         preferred_element_type=jnp.float32,
                         precision=lax.Precision.HIGHEST)
        out_ref[...] = logits / cnt[:, :2] + bc_ref[...]


def _node_pool(x, p0, p1, batch3d, W1, b12d, Wc, bc2d):
    return pl.pallas_call(
        _node_body,
        grid=(N // _BN,),
        in_specs=[
            pl.BlockSpec((_BN, D), lambda i: (i, 0)),
            pl.BlockSpec((_BN, D), lambda i: (i, 0)),
            pl.BlockSpec((_BN, D), lambda i: (i, 0)),
            pl.BlockSpec((1, 1, _BN), lambda i: (i, 0, 0)),
            pl.BlockSpec((D, H), lambda i: (0, 0)),
            pl.BlockSpec((1, H), lambda i: (0, 0)),
            pl.BlockSpec((H, 2), lambda i: (0, 0)),
            pl.BlockSpec((1, 2), lambda i: (0, 0)),
        ],
        out_specs=pl.BlockSpec((G, 2), lambda i: (0, 0)),
        out_shape=jax.ShapeDtypeStruct((G, 2), jnp.float32),
        scratch_shapes=[
            pltpu.VMEM((G, H), jnp.float32),
            pltpu.VMEM((G, G), jnp.float32),
        ],
    )(x, p0, p1, batch3d, W1, b12d, Wc, bc2d)


def kernel(x, edge_index, edge_attr, batch, We, be, W1, b1, Wc, bc):
    emb = _edge_emb(edge_attr, We, be.reshape(1, D))
    src = edge_index[0]
    dst = edge_index[1]
    partials = _make_sc_edge_kernel()(x, emb, src, dst)[:, :N]
    batch3d = batch.reshape(N // _BN, 1, _BN)
    return _node_pool(x, partials[0], partials[1], batch3d,
                      W1, b1.reshape(1, H), Wc, bc.reshape(1, 2))


# speedup vs baseline: 4.2918x; 1.1103x over previous
import functools

import jax
import jax.numpy as jnp
from jax import lax
from jax.experimental import pallas as pl
from jax.experimental.pallas import tpu as pltpu
from jax.experimental.pallas import tpu_sc as plsc

N = 10000
E = 320000
D = 128
DE = 16
H = 128
G = 128

NC = 2
NS = 16
NW = NC * NS
EP = E // NW
PASSES = 2
EPP = EP // PASSES
CH = 40
NCHUNK = EPP // CH
N_PAD = 10240
ROWS_PER_TILE = N_PAD // NS
ZR = CH


_BE = 4000


def _emb_body(ea_ref, we_ref, be_ref, out_ref):
    out_ref[...] = (
        jnp.dot(ea_ref[...], we_ref[...], preferred_element_type=jnp.float32)
        + be_ref[...]
    )


def _edge_emb(edge_attr, We, be2d):
    return pl.pallas_call(
        _emb_body,
        grid=(E // _BE,),
        in_specs=[
            pl.BlockSpec((_BE, DE), lambda i: (i, 0)),
            pl.BlockSpec((DE, D), lambda i: (0, 0)),
            pl.BlockSpec((1, D), lambda i: (0, 0)),
        ],
        out_specs=pl.BlockSpec((_BE, D), lambda i: (i, 0)),
        out_shape=jax.ShapeDtypeStruct((E, D), jnp.float32),
    )(edge_attr, We, be2d)


def _sc_edge_body(x_hbm, emb_hbm, src_hbm, dst_hbm, out_hbm,
                  agg_sh, sidx_all,
                  embv0, embv1, embv2, xv0, xv1, xv2, didx0, didx1, didx2,
                  sem_e0, sem_e1, sem_e2, sem_g0, sem_g1, sem_g2,
                  sem_c0, sem_c1, sem_c2, sem_d0, sem_d1, sem_d2):
    cid = lax.axis_index("c")
    sid = lax.axis_index("s")
    wid = cid * NS + sid
    base = wid * EP

    embv = (embv0, embv1, embv2)
    xv = (xv0, xv1, xv2)
    didx = (didx0, didx1, didx2)
    sem_e = (sem_e0, sem_e1, sem_e2)
    sem_g = (sem_g0, sem_g1, sem_g2)
    sem_c = (sem_c0, sem_c1, sem_c2)
    sem_d = (sem_d0, sem_d1, sem_d2)

    def _issue_inputs(pbase, j, p):
        pltpu.async_copy(emb_hbm.at[pl.ds(pbase + j * CH, CH)], embv[p], sem_e[p])
        pltpu.async_copy(x_hbm.at[sidx_all.at[pl.ds(j * CH, CH)]], xv[p], sem_g[p])
        pltpu.async_copy(dst_hbm.at[pl.ds(pbase + j * CH, CH)], didx[p], sem_d[p])

    def _wait_inputs(p):
        pltpu.make_async_copy(emb_hbm.at[pl.ds(base, CH)], embv[p], sem_e[p]).wait()
        pltpu.make_async_copy(x_hbm.at[sidx_all.at[pl.ds(0, CH)]], xv[p], sem_g[p]).wait()
        pltpu.make_async_copy(dst_hbm.at[pl.ds(base, CH)], didx[p], sem_d[p]).wait()

    def _issue_scatter(j, p):
        pltpu.async_copy(embv[p], agg_sh.at[didx[p]], sem_c[p], add=True)

    def _wait_scatter(p):
        pltpu.make_async_copy(embv[p], agg_sh.at[didx[p]], sem_c[p]).wait()

    def _compute(p):
        ev = embv[p]
        gv = xv[p]

        @plsc.parallel_loop(0, CH, step=1, unroll=4)
        def _row(r):
            for c in range(D // 16):
                sl = pl.ds(c * 16, 16)
                ev[r, sl] = jnp.maximum(gv[r, sl] + ev[r, sl], 0.0)

    NT = (NCHUNK - 2) // 3

    for q in range(PASSES):
        pbase = base + q * EPP
        pltpu.sync_copy(src_hbm.at[pl.ds(pbase, EPP)], sidx_all)
        _issue_inputs(pbase, 0, 0)
        _issue_inputs(pbase, 1, 1)

        if q == 0:
            zeros16 = jnp.zeros((16,), jnp.float32)

            def _zrow(r, _):
                for c in range(D // 16):
                    embv2[r, pl.ds(c * 16, 16)] = zeros16
                return 0

            lax.fori_loop(0, ZR, _zrow, 0)
            for k in range(ROWS_PER_TILE // ZR):
                pltpu.sync_copy(
                    embv2, agg_sh.at[pl.ds(sid * ROWS_PER_TILE + k * ZR, ZR)])
            plsc.subcore_barrier()

        def _step(j, p, guard_first):
            _wait_inputs(p)
            _compute(p)
            if guard_first:
                pl.when(j >= 1)(lambda: _wait_scatter((p + 2) % 3))
            else:
                _wait_scatter((p + 2) % 3)
            _issue_inputs(pbase, j + 2, (p + 2) % 3)
            _issue_scatter(j, p)

        def _triple(t, _):
            j0 = 3 * t
            _step(j0, 0, True)
            _step(j0 + 1, 1, False)
            _step(j0 + 2, 2, False)
            return 0

        lax.fori_loop(0, NT, _triple, 0)

        for j in range(3 * NT, NCHUNK):
            p = j % 3
            _wait_inputs(p)
            _compute(p)
            _wait_scatter((p + 2) % 3)
            _issue_scatter(j, p)
        _wait_scatter((NCHUNK - 1) % 3)
    plsc.subcore_barrier()

    row0 = sid * ROWS_PER_TILE
    pltpu.sync_copy(agg_sh.at[pl.ds(row0, ROWS_PER_TILE)],
                    out_hbm.at[cid, pl.ds(row0, ROWS_PER_TILE)])


@functools.cache
def _make_sc_edge_kernel():
    return functools.partial(
        pl.kernel,
        out_type=jax.ShapeDtypeStruct((NC, N_PAD, D), jnp.float32),
        mesh=plsc.VectorSubcoreMesh(core_axis_name="c", subcore_axis_name="s",
                                    num_cores=NC, num_subcores=NS),
        scratch_types=[
            pltpu.VMEM_SHARED((N_PAD, D), jnp.float32),
            pltpu.VMEM((EPP,), jnp.int32),
            pltpu.VMEM((CH, D), jnp.float32),
            pltpu.VMEM((CH, D), jnp.float32),
            pltpu.VMEM((CH, D), jnp.float32),
            pltpu.VMEM((CH, D), jnp.float32),
            pltpu.VMEM((CH, D), jnp.float32),
            pltpu.VMEM((CH, D), jnp.float32),
            pltpu.VMEM((CH,), jnp.int32),
            pltpu.VMEM((CH,), jnp.int32),
            pltpu.VMEM((CH,), jnp.int32),
            pltpu.SemaphoreType.DMA,
            pltpu.SemaphoreType.DMA,
            pltpu.SemaphoreType.DMA,
            pltpu.SemaphoreType.DMA,
            pltpu.SemaphoreType.DMA,
            pltpu.SemaphoreType.DMA,
            pltpu.SemaphoreType.DMA,
            pltpu.SemaphoreType.DMA,
            pltpu.SemaphoreType.DMA,
            pltpu.SemaphoreType.DMA,
            pltpu.SemaphoreType.DMA,
            pltpu.SemaphoreType.DMA,
        ],
    )(_sc_edge_body)


_BN = 2000


def _node_body(x_ref, p0_ref, p1_ref, b_ref, w1_ref, b1_ref, wc_ref, bc_ref,
               out_ref, pooled_acc, cnt_acc):
    i = pl.program_id(0)

    @pl.when(i == 0)
    def _():
        pooled_acc[...] = jnp.zeros_like(pooled_acc)
        cnt_acc[...] = jnp.zeros_like(cnt_acc)

    t = x_ref[...] + p0_ref[...] + p1_ref[...]
    h = jnp.maximum(
        jnp.dot(t, w1_ref[...], preferred_element_type=jnp.float32,
                precision=lax.Precision.HIGHEST) + b1_ref[...],
        0.0,
    )
    bvec = b_ref[0, 0, :]
    gids = lax.broadcasted_iota(jnp.int32, (_BN, G), 1)
    onehot = jnp.where(bvec[:, None] == gids, 1.0, 0.0)
    pooled_acc[...] += lax.dot_general(
        onehot, h, (((0,), (0,)), ((), ())), preferred_element_type=jnp.float32,
        precision=lax.Precision.HIGHEST)
    cnt_acc[...] += lax.dot_general(
        onehot, jnp.ones((_BN, G), jnp.float32), (((0,), (0,)), ((), ())),
        preferred_element_type=jnp.float32)

    @pl.when(i == pl.num_programs(0) - 1)
    def _():
        cnt = jnp.maximum(cnt_acc[...], 1.0)
        logits = jnp.dot(pooled_acc[...], wc_ref[...],
                         preferred_element_type=jnp.float32,
                         precision=lax.Precision.HIGHEST)
        out_ref[...] = logits / cnt[:, :2] + bc_ref[...]


def _node_pool(x, p0, p1, batch3d, W1, b12d, Wc, bc2d):
    return pl.pallas_call(
        _node_body,
        grid=(N // _BN,),
        in_specs=[
            pl.BlockSpec((_BN, D), lambda i: (i, 0)),
            pl.BlockSpec((_BN, D), lambda i: (i, 0)),
            pl.BlockSpec((_BN, D), lambda i: (i, 0)),
            pl.BlockSpec((1, 1, _BN), lambda i: (i, 0, 0)),
            pl.BlockSpec((D, H), lambda i: (0, 0)),
            pl.BlockSpec((1, H), lambda i: (0, 0)),
            pl.BlockSpec((H, 2), lambda i: (0, 0)),
            pl.BlockSpec((1, 2), lambda i: (0, 0)),
        ],
        out_specs=pl.BlockSpec((G, 2), lambda i: (0, 0)),
        out_shape=jax.ShapeDtypeStruct((G, 2), jnp.float32),
        scratch_shapes=[
            pltpu.VMEM((G, H), jnp.float32),
            pltpu.VMEM((G, G), jnp.float32),
        ],
    )(x, p0, p1, batch3d, W1, b12d, Wc, bc2d)


def kernel(x, edge_index, edge_attr, batch, We, be, W1, b1, Wc, bc):
    emb = _edge_emb(edge_attr, We, be.reshape(1, D))
    src = edge_index[0]
    dst = edge_index[1]
    partials = _make_sc_edge_kernel()(x, emb, src, dst)[:, :N]
    batch3d = batch.reshape(N // _BN, 1, _BN)
    return _node_pool(x, partials[0], partials[1], batch3d,
                      W1, b1.reshape(1, H), Wc, bc.reshape(1, 2))


# speedup vs baseline: 6.1263x; 1.4275x over previous
import functools

import jax
import jax.numpy as jnp
from jax import lax
from jax.experimental import pallas as pl
from jax.experimental.pallas import tpu as pltpu
from jax.experimental.pallas import tpu_sc as plsc

N = 10000
E = 320000
D = 128
DE = 16
H = 128
G = 128

NC = 2
NS = 16
NW = NC * NS
EP = E // NW
PASSES = 2
EPP = EP // PASSES
CH = 40
NCHUNK = EPP // CH
N_PAD = 10240
ROWS_PER_TILE = N_PAD // NS
ZR = CH


_BE = 16000


def _emb_body(eat_ref, we_ref, be_ref, out_ref):
    out_ref[...] = (
        lax.dot_general(eat_ref[...], we_ref[...], (((0,), (0,)), ((), ())),
                        preferred_element_type=jnp.float32)
        + be_ref[...]
    )


def _edge_emb(edge_attr_t, We, be2d):
    return pl.pallas_call(
        _emb_body,
        grid=(E // _BE,),
        in_specs=[
            pl.BlockSpec((DE, _BE), lambda i: (0, i)),
            pl.BlockSpec((DE, D), lambda i: (0, 0)),
            pl.BlockSpec((1, D), lambda i: (0, 0)),
        ],
        out_specs=pl.BlockSpec((_BE, D), lambda i: (i, 0)),
        out_shape=jax.ShapeDtypeStruct((E, D), jnp.float32),
    )(edge_attr_t, We, be2d)


def _sc_edge_body(x_hbm, emb_hbm, src_hbm, dst_hbm, out_hbm,
                  agg_sh, sidx_all,
                  embv0, embv1, embv2, xv0, xv1, xv2, didx0, didx1, didx2,
                  sem_e0, sem_e1, sem_e2, sem_g0, sem_g1, sem_g2,
                  sem_c0, sem_c1, sem_c2, sem_d0, sem_d1, sem_d2):
    cid = lax.axis_index("c")
    sid = lax.axis_index("s")
    wid = cid * NS + sid
    base = wid * EP

    embv = (embv0, embv1, embv2)
    xv = (xv0, xv1, xv2)
    didx = (didx0, didx1, didx2)
    sem_e = (sem_e0, sem_e1, sem_e2)
    sem_g = (sem_g0, sem_g1, sem_g2)
    sem_c = (sem_c0, sem_c1, sem_c2)
    sem_d = (sem_d0, sem_d1, sem_d2)

    def _issue_inputs(pbase, j, p):
        pltpu.async_copy(emb_hbm.at[pl.ds(pbase + j * CH, CH)], embv[p], sem_e[p])
        pltpu.async_copy(x_hbm.at[sidx_all.at[pl.ds(j * CH, CH)]], xv[p], sem_g[p])
        pltpu.async_copy(dst_hbm.at[pl.ds(pbase + j * CH, CH)], didx[p], sem_d[p])

    def _wait_inputs(p):
        pltpu.make_async_copy(emb_hbm.at[pl.ds(base, CH)], embv[p], sem_e[p]).wait()
        pltpu.make_async_copy(x_hbm.at[sidx_all.at[pl.ds(0, CH)]], xv[p], sem_g[p]).wait()
        pltpu.make_async_copy(dst_hbm.at[pl.ds(base, CH)], didx[p], sem_d[p]).wait()

    def _issue_scatter(j, p):
        pltpu.async_copy(embv[p], agg_sh.at[didx[p]], sem_c[p], add=True)

    def _wait_scatter(p):
        pltpu.make_async_copy(embv[p], agg_sh.at[didx[p]], sem_c[p]).wait()

    def _compute(p):
        ev = embv[p]
        gv = xv[p]

        @plsc.parallel_loop(0, CH, step=1, unroll=4)
        def _row(r):
            for c in range(D // 16):
                sl = pl.ds(c * 16, 16)
                ev[r, sl] = jnp.maximum(gv[r, sl] + ev[r, sl], 0.0)

    NT = (NCHUNK - 2) // 3

    for q in range(PASSES):
        pbase = base + q * EPP
        pltpu.sync_copy(src_hbm.at[pl.ds(pbase, EPP)], sidx_all)
        _issue_inputs(pbase, 0, 0)
        _issue_inputs(pbase, 1, 1)

        if q == 0:
            zeros16 = jnp.zeros((16,), jnp.float32)

            def _zrow(r, _):
                for c in range(D // 16):
                    embv2[r, pl.ds(c * 16, 16)] = zeros16
                return 0

            lax.fori_loop(0, ZR, _zrow, 0)
            for k in range(ROWS_PER_TILE // ZR):
                pltpu.sync_copy(
                    embv2, agg_sh.at[pl.ds(sid * ROWS_PER_TILE + k * ZR, ZR)])
            plsc.subcore_barrier()

        def _step(j, p, guard_first):
            _wait_inputs(p)
            _compute(p)
            if guard_first:
                pl.when(j >= 1)(lambda: _wait_scatter((p + 2) % 3))
            else:
                _wait_scatter((p + 2) % 3)
            _issue_inputs(pbase, j + 2, (p + 2) % 3)
            _issue_scatter(j, p)

        def _triple(t, _):
            j0 = 3 * t
            _step(j0, 0, True)
            _step(j0 + 1, 1, False)
            _step(j0 + 2, 2, False)
            return 0

        lax.fori_loop(0, NT, _triple, 0)

        for j in range(3 * NT, NCHUNK):
            p = j % 3
            _wait_inputs(p)
            _compute(p)
            _wait_scatter((p + 2) % 3)
            _issue_scatter(j, p)
        _wait_scatter((NCHUNK - 1) % 3)
    plsc.subcore_barrier()

    row0 = sid * ROWS_PER_TILE
    pltpu.sync_copy(agg_sh.at[pl.ds(row0, ROWS_PER_TILE)],
                    out_hbm.at[cid, pl.ds(row0, ROWS_PER_TILE)])


@functools.cache
def _make_sc_edge_kernel():
    return functools.partial(
        pl.kernel,
        out_type=jax.ShapeDtypeStruct((NC, N_PAD, D), jnp.float32),
        mesh=plsc.VectorSubcoreMesh(core_axis_name="c", subcore_axis_name="s",
                                    num_cores=NC, num_subcores=NS),
        scratch_types=[
            pltpu.VMEM_SHARED((N_PAD, D), jnp.float32),
            pltpu.VMEM((EPP,), jnp.int32),
            pltpu.VMEM((CH, D), jnp.float32),
            pltpu.VMEM((CH, D), jnp.float32),
            pltpu.VMEM((CH, D), jnp.float32),
            pltpu.VMEM((CH, D), jnp.float32),
            pltpu.VMEM((CH, D), jnp.float32),
            pltpu.VMEM((CH, D), jnp.float32),
            pltpu.VMEM((CH,), jnp.int32),
            pltpu.VMEM((CH,), jnp.int32),
            pltpu.VMEM((CH,), jnp.int32),
            pltpu.SemaphoreType.DMA,
            pltpu.SemaphoreType.DMA,
            pltpu.SemaphoreType.DMA,
            pltpu.SemaphoreType.DMA,
            pltpu.SemaphoreType.DMA,
            pltpu.SemaphoreType.DMA,
            pltpu.SemaphoreType.DMA,
            pltpu.SemaphoreType.DMA,
            pltpu.SemaphoreType.DMA,
            pltpu.SemaphoreType.DMA,
            pltpu.SemaphoreType.DMA,
            pltpu.SemaphoreType.DMA,
        ],
    )(_sc_edge_body)


_BN = 2000


def _node_body(x_ref, p0_ref, p1_ref, b_ref, w1_ref, b1_ref, wc_ref, bc_ref,
               out_ref, pooled_acc, cnt_acc):
    i = pl.program_id(0)

    @pl.when(i == 0)
    def _():
        pooled_acc[...] = jnp.zeros_like(pooled_acc)
        cnt_acc[...] = jnp.zeros_like(cnt_acc)

    t = x_ref[...] + p0_ref[...] + p1_ref[...]
    h = jnp.maximum(
        jnp.dot(t, w1_ref[...], preferred_element_type=jnp.float32,
                precision=lax.Precision.HIGHEST) + b1_ref[...],
        0.0,
    )
    bvec = b_ref[0, 0, :]
    gids = lax.broadcasted_iota(jnp.int32, (_BN, G), 1)
    onehot = jnp.where(bvec[:, None] == gids, 1.0, 0.0)
    pooled_acc[...] += lax.dot_general(
        onehot, h, (((0,), (0,)), ((), ())), preferred_element_type=jnp.float32,
        precision=lax.Precision.HIGHEST)
    cnt_acc[...] += lax.dot_general(
        onehot, jnp.ones((_BN, G), jnp.float32), (((0,), (0,)), ((), ())),
        preferred_element_type=jnp.float32)

    @pl.when(i == pl.num_programs(0) - 1)
    def _():
        cnt = jnp.maximum(cnt_acc[...], 1.0)
        logits = jnp.dot(pooled_acc[...], wc_ref[...],
                         preferred_element_type=jnp.float32,
                         precision=lax.Precision.HIGHEST)
        out_ref[...] = logits / cnt[:, :2] + bc_ref[...]


def _node_pool(x, p0, p1, batch3d, W1, b12d, Wc, bc2d):
    return pl.pallas_call(
        _node_body,
        grid=(N // _BN,),
        in_specs=[
            pl.BlockSpec((_BN, D), lambda i: (i, 0)),
            pl.BlockSpec((_BN, D), lambda i: (i, 0)),
            pl.BlockSpec((_BN, D), lambda i: (i, 0)),
            pl.BlockSpec((1, 1, _BN), lambda i: (i, 0, 0)),
            pl.BlockSpec((D, H), lambda i: (0, 0)),
            pl.BlockSpec((1, H), lambda i: (0, 0)),
            pl.BlockSpec((H, 2), lambda i: (0, 0)),
            pl.BlockSpec((1, 2), lambda i: (0, 0)),
        ],
        out_specs=pl.BlockSpec((G, 2), lambda i: (0, 0)),
        out_shape=jax.ShapeDtypeStruct((G, 2), jnp.float32),
        scratch_shapes=[
            pltpu.VMEM((G, H), jnp.float32),
            pltpu.VMEM((G, G), jnp.float32),
        ],
    )(x, p0, p1, batch3d, W1, b12d, Wc, bc2d)


def kernel(x, edge_index, edge_attr, batch, We, be, W1, b1, Wc, bc):
    emb = _edge_emb(edge_attr.T, We, be.reshape(1, D))
    src = edge_index[0]
    dst = edge_index[1]
    partials = _make_sc_edge_kernel()(x, emb, src, dst)[:, :N]
    batch3d = batch.reshape(N // _BN, 1, _BN)
    return _node_pool(x, partials[0], partials[1], batch3d,
                      W1, b1.reshape(1, H), Wc, bc.reshape(1, 2))
